# baseline (speedup 1.0000x reference)
import jax
import jax.numpy as jnp
from jax.experimental import pallas as pl


def _add_body(patches_ref, pos_ref, out_ref):
    out_ref[...] = patches_ref[...] + pos_ref[...]


def kernel(patches, pos_table):
    batch, n_patches, model_dim = patches.shape
    return pl.pallas_call(
        _add_body,
        grid=(batch,),
        in_specs=[
            pl.BlockSpec((1, n_patches, model_dim), lambda b: (b, 0, 0)),
            pl.BlockSpec((n_patches, model_dim), lambda b: (0, 0)),
        ],
        out_specs=pl.BlockSpec((1, n_patches, model_dim), lambda b: (b, 0, 0)),
        out_shape=jax.ShapeDtypeStruct(patches.shape, patches.dtype),
    )(patches, pos_table)


# speedup vs baseline: 1.1848x; 1.1848x over previous
import jax
import jax.numpy as jnp
from jax.experimental import pallas as pl


def _add_body(patches_ref, pos_ref, out_ref):
    out_ref[...] = patches_ref[...] + pos_ref[...]


def kernel(patches, pos_table):
    batch, n_patches, model_dim = patches.shape
    bb = 4
    return pl.pallas_call(
        _add_body,
        grid=(batch // bb,),
        in_specs=[
            pl.BlockSpec((bb, n_patches, model_dim), lambda b: (b, 0, 0)),
            pl.BlockSpec((n_patches, model_dim), lambda b: (0, 0)),
        ],
        out_specs=pl.BlockSpec((bb, n_patches, model_dim), lambda b: (b, 0, 0)),
        out_shape=jax.ShapeDtypeStruct(patches.shape, patches.dtype),
    )(patches, pos_table)


# speedup vs baseline: 1.2065x; 1.0183x over previous
import jax
import jax.numpy as jnp
from jax.experimental import pallas as pl


def _add_body(patches_ref, pos_ref, out_ref):
    out_ref[...] = patches_ref[...] + pos_ref[...]


def kernel(patches, pos_table):
    batch, n_patches, model_dim = patches.shape
    bb = 8
    return pl.pallas_call(
        _add_body,
        grid=(batch // bb,),
        in_specs=[
            pl.BlockSpec((bb, n_patches, model_dim), lambda b: (b, 0, 0)),
            pl.BlockSpec((n_patches, model_dim), lambda b: (0, 0)),
        ],
        out_specs=pl.BlockSpec((bb, n_patches, model_dim), lambda b: (b, 0, 0)),
        out_shape=jax.ShapeDtypeStruct(patches.shape, patches.dtype),
    )(patches, pos_table)
